# baseline (speedup 1.0000x reference)
import functools

import jax
import jax.numpy as jnp
from jax import lax
from jax.experimental import pallas as pl
from jax.experimental.pallas import tpu as pltpu
from jax.experimental.pallas import tpu_sc as plsc

MAX_DIST = 32
TABLE_SIZE = 2 * MAX_DIST + 1
VALS = MAX_DIST + 1
K = 128
NWORK = 32


def _sc_onehot(pd, ph, pw):
    R = pd.shape[0]
    rows = R // NWORK
    mesh = plsc.VectorSubcoreMesh(core_axis_name="c", subcore_axis_name="s")

    @functools.partial(
        pl.kernel, mesh=mesh,
        out_type=jax.ShapeDtypeStruct((R, K), jnp.float32),
        compiler_params=pltpu.CompilerParams(needs_layout_passes=False),
        scratch_types=[
            pltpu.VMEM((rows,), jnp.int32),
            pltpu.VMEM((rows,), jnp.int32),
            pltpu.VMEM((rows,), jnp.int32),
            pltpu.VMEM((rows, K), jnp.float32),
        ],
    )
    def enc(pd_hbm, ph_hbm, pw_hbm, out_hbm, pd_v, ph_v, pw_v, buf):
        wid = lax.axis_index("s") * 2 + lax.axis_index("c")
        base = wid * rows
        pltpu.sync_copy(pd_hbm.at[pl.ds(base, rows)], pd_v)
        pltpu.sync_copy(ph_hbm.at[pl.ds(base, rows)], ph_v)
        pltpu.sync_copy(pw_hbm.at[pl.ds(base, rows)], pw_v)

        kvecs = [lax.iota(jnp.int32, 16) + 16 * blk for blk in range(K // 16)]

        def row_body(r, _):
            ivec = lax.broadcast_in_dim(r, (16,), ())
            pdb = plsc.load_gather(pd_v, [ivec])
            phb = plsc.load_gather(ph_v, [ivec]) + VALS
            pwb = plsc.load_gather(pw_v, [ivec]) + 2 * VALS
            for blk in range(K // 16):
                kv = kvecs[blk]
                hit = (kv == pdb) | (kv == phb) | (kv == pwb)
                buf[r, pl.ds(blk * 16, 16)] = jnp.where(hit, 1.0, 0.0)
            return ()

        lax.fori_loop(0, rows, row_body, ())
        pltpu.sync_copy(buf, out_hbm.at[pl.ds(base, rows)])

    return enc(pd, ph, pw)


def _bias_kernel(o_all_ref, m_ref, out_ref, *, hb):
    of = o_all_ref[0]
    for hh in range(hb):
        m = m_ref[hh].astype(jnp.bfloat16)
        a = jnp.dot(of, m, preferred_element_type=jnp.float32)
        out = jax.lax.dot_general(
            a.astype(jnp.bfloat16), of, (((1,), (1,)), ((), ())),
            preferred_element_type=jnp.float32)
        out_ref[0, hh] = out


@functools.partial(jax.jit, static_argnames=())
def kernel(positions, rel_bias_d, rel_bias_h, rel_bias_w):
    B, N, _ = positions.shape
    H = rel_bias_d.shape[1]
    HB = 2

    pos = jnp.clip(positions.astype(jnp.int32), 0, MAX_DIST)
    onehot = _sc_onehot(
        pos[..., 0].reshape(-1), pos[..., 1].reshape(-1),
        pos[..., 2].reshape(-1))
    onehot = onehot.astype(jnp.bfloat16).reshape(B, N, K)

    u = jnp.arange(VALS, dtype=jnp.int32)
    duv = u[:, None] - u[None, :] + MAX_DIST
    md = rel_bias_d[duv].transpose(2, 0, 1)
    mh = rel_bias_h[duv].transpose(2, 0, 1)
    mw = rel_bias_w[duv].transpose(2, 0, 1)
    m = jnp.zeros((H, K, K), dtype=jnp.float32)
    m = m.at[:, 0:VALS, 0:VALS].set(md)
    m = m.at[:, VALS:2 * VALS, VALS:2 * VALS].set(mh)
    m = m.at[:, 2 * VALS:3 * VALS, 2 * VALS:3 * VALS].set(mw)

    grid = (B, H // HB)
    out = pl.pallas_call(
        functools.partial(_bias_kernel, hb=HB),
        grid=grid,
        in_specs=[
            pl.BlockSpec((1, N, K), lambda b, hg: (b, 0, 0)),
            pl.BlockSpec((HB, K, K), lambda b, hg: (hg, 0, 0)),
        ],
        out_specs=pl.BlockSpec((1, HB, N, N), lambda b, hg: (b, hg, 0, 0)),
        out_shape=jax.ShapeDtypeStruct((B, H, N, N), jnp.float32),
    )(onehot, m)
    return out


# speedup vs baseline: 1.0102x; 1.0102x over previous
import functools

import jax
import jax.numpy as jnp
from jax import lax
from jax.experimental import pallas as pl
from jax.experimental.pallas import tpu as pltpu
from jax.experimental.pallas import tpu_sc as plsc

MAX_DIST = 32
TABLE_SIZE = 2 * MAX_DIST + 1
VALS = MAX_DIST + 1
K = 128
NWORK = 32


def _sc_onehot(pd, ph, pw):
    R = pd.shape[0]
    rows = R // NWORK
    mesh = plsc.VectorSubcoreMesh(core_axis_name="c", subcore_axis_name="s")

    @functools.partial(
        pl.kernel, mesh=mesh,
        out_type=jax.ShapeDtypeStruct((R, K), jnp.float32),
        compiler_params=pltpu.CompilerParams(needs_layout_passes=False),
        scratch_types=[
            pltpu.VMEM((rows,), jnp.int32),
            pltpu.VMEM((rows,), jnp.int32),
            pltpu.VMEM((rows,), jnp.int32),
            pltpu.VMEM((rows, K), jnp.float32),
        ],
    )
    def enc(pd_hbm, ph_hbm, pw_hbm, out_hbm, pd_v, ph_v, pw_v, buf):
        wid = lax.axis_index("s") * 2 + lax.axis_index("c")
        base = wid * rows
        pltpu.sync_copy(pd_hbm.at[pl.ds(base, rows)], pd_v)
        pltpu.sync_copy(ph_hbm.at[pl.ds(base, rows)], ph_v)
        pltpu.sync_copy(pw_hbm.at[pl.ds(base, rows)], pw_v)

        kvecs = [lax.iota(jnp.int32, 16) + 16 * blk for blk in range(K // 16)]

        def row_body(r, _):
            ivec = lax.broadcast_in_dim(r, (16,), ())
            pdb = plsc.load_gather(pd_v, [ivec])
            phb = plsc.load_gather(ph_v, [ivec]) + VALS
            pwb = plsc.load_gather(pw_v, [ivec]) + 2 * VALS
            for blk in range(K // 16):
                kv = kvecs[blk]
                hit = (kv == pdb) | (kv == phb) | (kv == pwb)
                buf[r, pl.ds(blk * 16, 16)] = jnp.where(hit, 1.0, 0.0)
            return ()

        lax.fori_loop(0, rows, row_body, ())
        pltpu.sync_copy(buf, out_hbm.at[pl.ds(base, rows)])

    return enc(pd, ph, pw)


def _bias_kernel(o_all_ref, m_ref, out_ref, *, hb):
    of = o_all_ref[0].astype(jnp.bfloat16)
    for hh in range(hb):
        m = m_ref[hh].astype(jnp.bfloat16)
        a = jnp.dot(of, m, preferred_element_type=jnp.float32)
        out = jax.lax.dot_general(
            a.astype(jnp.bfloat16), of, (((1,), (1,)), ((), ())),
            preferred_element_type=jnp.float32)
        out_ref[0, hh] = out


@functools.partial(jax.jit, static_argnames=())
def kernel(positions, rel_bias_d, rel_bias_h, rel_bias_w):
    B, N, _ = positions.shape
    H = rel_bias_d.shape[1]
    HB = 2

    pos = jnp.clip(positions.astype(jnp.int32), 0, MAX_DIST)
    onehot = _sc_onehot(
        pos[..., 0].reshape(-1), pos[..., 1].reshape(-1),
        pos[..., 2].reshape(-1))
    onehot = onehot.reshape(B, N, K)

    u = jnp.arange(VALS, dtype=jnp.int32)
    duv = u[:, None] - u[None, :] + MAX_DIST
    md = rel_bias_d[duv].transpose(2, 0, 1)
    mh = rel_bias_h[duv].transpose(2, 0, 1)
    mw = rel_bias_w[duv].transpose(2, 0, 1)
    m = jnp.zeros((H, K, K), dtype=jnp.float32)
    m = m.at[:, 0:VALS, 0:VALS].set(md)
    m = m.at[:, VALS:2 * VALS, VALS:2 * VALS].set(mh)
    m = m.at[:, 2 * VALS:3 * VALS, 2 * VALS:3 * VALS].set(mw)

    grid = (B, H // HB)
    out = pl.pallas_call(
        functools.partial(_bias_kernel, hb=HB),
        grid=grid,
        in_specs=[
            pl.BlockSpec((1, N, K), lambda b, hg: (b, 0, 0)),
            pl.BlockSpec((HB, K, K), lambda b, hg: (hg, 0, 0)),
        ],
        out_specs=pl.BlockSpec((1, HB, N, N), lambda b, hg: (b, hg, 0, 0)),
        out_shape=jax.ShapeDtypeStruct((B, H, N, N), jnp.float32),
    )(onehot, m)
    return out
